# baseline (speedup 1.0000x reference)
import functools

import jax
import jax.numpy as jnp
from jax import lax
from jax.experimental import pallas as pl
from jax.experimental.pallas import tpu as pltpu
from jax.experimental.pallas import tpu_sc as plsc

N_EGO = 10000
N_VEH = 40000
E = 250000
HID = 128
B = 64

PAD_ROWS = 64
EPAD = 262144
PADDED = {N_EGO: 10112, N_VEH: 40064}
NB = 4
ROWB = 1000
INT_MIN = -2147483648
_INV_SQRT2 = 0.7071067811865476


def _gelu(x):
    return 0.5 * x * (1.0 + lax.erf(x * _INV_SQRT2))


def _ln(x, g, b):
    mu = jnp.mean(x, axis=1, keepdims=True)
    xc = x - mu
    var = jnp.mean(xc * xc, axis=1, keepdims=True)
    return xc * lax.rsqrt(var + 1e-5) * g + b


@functools.lru_cache(maxsize=None)
def _make_sc_agg(n_src, n_dst, dc, ppc):
    n_src_pad = PADDED[n_src]
    n_dst_pad = PADDED[n_dst]
    rpt = n_dst_pad // 16
    spt = n_src_pad // 16
    n_q = 2 * ppc
    chunks = 128
    mesh = plsc.VectorSubcoreMesh(
        core_axis_name="c", subcore_axis_name="s", num_cores=2, num_subcores=16
    )
    out_type = tuple(
        jax.ShapeDtypeStruct((n_dst_pad, dc), jnp.float32) for _ in range(n_q)
    )
    scratch = [
        pltpu.VMEM((chunks * 128,), jnp.int32),
        pltpu.VMEM((chunks, 128), jnp.int32),
        pltpu.VMEM((NB, 128, dc), jnp.float32),
        pltpu.VMEM_SHARED((n_dst_pad, dc), jnp.float32),
        pltpu.VMEM_SHARED((n_src_pad, dc), jnp.float32),
        pltpu.SemaphoreType.DMA((NB,)),
        pltpu.SemaphoreType.DMA((NB,)),
    ]

    def body(table, srcI, dstI, zeros, *rest):
        outs = rest[:n_q]
        sidx, didx, gbuf, acc, tabs, gsem, ssem = rest[n_q:]
        c = lax.axis_index("c")
        s = lax.axis_index("s")
        pltpu.sync_copy(srcI.at[pl.ds(s * (chunks * 128), chunks * 128)], sidx)
        pltpu.sync_copy(dstI.at[pl.ds(s * chunks, chunks)], didx)

        for p in range(ppc):
            q = c * ppc + p
            pltpu.sync_copy(zeros, acc.at[pl.ds(s * rpt, rpt)])
            pltpu.sync_copy(
                table.at[pl.ds(q * n_src_pad + s * spt, spt)],
                tabs.at[pl.ds(s * spt, spt)],
            )
            plsc.subcore_barrier()

            def g_copy(k, b):
                return pltpu.make_async_copy(
                    tabs.at[sidx.at[pl.ds(k * 128, 128)]], gbuf.at[b],
                    gsem.at[b],
                )

            def s_copy(k, b):
                return pltpu.make_async_copy(
                    gbuf.at[b], acc.at[didx.at[k]], ssem.at[b]
                )

            for b in range(NB):
                g_copy(b, b).start()

            def outer(j, _):
                for b in range(NB):
                    k = j * NB + b
                    g_copy(k, b).wait()
                    s_copy(k, b).start(add=True)
                for b in range(NB):
                    k = j * NB + b
                    s_copy(k, b).wait()

                    @pl.when(k + NB < chunks)
                    def _(k=k, b=b):
                        g_copy(k + NB, b).start()

                return 0

            lax.fori_loop(0, chunks // NB, outer, 0)
            plsc.subcore_barrier()

            for qo in range(n_q):
                if qo % ppc != p:
                    continue

                @pl.when(c == qo // ppc)
                def _(qo=qo):
                    pltpu.sync_copy(
                        acc.at[pl.ds(s * rpt, rpt)],
                        outs[qo].at[pl.ds(s * rpt, rpt)],
                    )

    return pl.kernel(
        body, out_type=out_type, mesh=mesh, scratch_types=scratch,
        compiler_params=pltpu.CompilerParams(use_tc_tiling_on_sc=False),
    )


@functools.lru_cache(maxsize=None)
def _make_sc_count(n_dst):
    n_dst_pad = PADDED[n_dst]
    rpt = n_dst_pad // 16
    chunks = 64
    mesh = plsc.VectorSubcoreMesh(
        core_axis_name="c", subcore_axis_name="s", num_cores=2, num_subcores=16
    )
    out_type = (
        jax.ShapeDtypeStruct((n_dst_pad, 16), jnp.float32),
        jax.ShapeDtypeStruct((n_dst_pad, 16), jnp.float32),
    )
    scratch = [
        pltpu.VMEM((chunks, 128), jnp.int32),
        pltpu.VMEM((128, 16), jnp.float32),
        pltpu.VMEM_SHARED((n_dst_pad, 16), jnp.float32),
        pltpu.SemaphoreType.DMA((NB,)),
    ]

    def body(ones, dstI, zeros, out0, out1, didx, gbuf, acc, ssem):
        c = lax.axis_index("c")
        s = lax.axis_index("s")
        w = c * 16 + s
        pltpu.sync_copy(dstI.at[pl.ds(w * chunks, chunks)], didx)
        pltpu.sync_copy(ones, gbuf)
        pltpu.sync_copy(zeros, acc.at[pl.ds(s * rpt, rpt)])
        plsc.subcore_barrier()

        def s_copy(k, b):
            return pltpu.make_async_copy(gbuf, acc.at[didx.at[k]], ssem.at[b])

        def outer(j, _):
            for b in range(NB):
                s_copy(j * NB + b, b).start(add=True)
            for b in range(NB):
                s_copy(j * NB + b, b).wait()
            return 0

        lax.fori_loop(0, chunks // NB, outer, 0)
        plsc.subcore_barrier()

        outs = (out0, out1)
        for oc in range(2):
            @pl.when(c == oc)
            def _(oc=oc):
                pltpu.sync_copy(
                    acc.at[pl.ds(s * rpt, rpt)],
                    outs[oc].at[pl.ds(s * rpt, rpt)],
                )

    return pl.kernel(
        body, out_type=out_type, mesh=mesh, scratch_types=scratch,
        compiler_params=pltpu.CompilerParams(use_tc_tiling_on_sc=False),
    )


def _pad_edges(src, dst, n_src, n_dst):
    pad = EPAD - E
    ar = jnp.arange(pad, dtype=jnp.int32)
    src_p = jnp.concatenate([src.astype(jnp.int32), ar % n_src])
    dst_p = jnp.concatenate(
        [dst.astype(jnp.int32), n_dst + (ar % PAD_ROWS)]
    )
    return src_p, dst_p.reshape(EPAD // 128, 128)


def _enc_body(x_ref, g_ref, b_ref, w1_ref, b1_ref, w2_ref, b2_ref, o_ref):
    x = x_ref[...]
    xn = _ln(x, g_ref[...], b_ref[...])
    h = jnp.dot(xn, w1_ref[...], preferred_element_type=jnp.float32) + b1_ref[...]
    h = _gelu(h)
    o_ref[...] = (
        jnp.dot(h, w2_ref[...], preferred_element_type=jnp.float32) + b2_ref[...]
    )


def _enc(x, p, n):
    r2 = lambda a: a.reshape(1, -1)
    full = lambda shape: pl.BlockSpec(shape, lambda i: (0, 0))
    return pl.pallas_call(
        _enc_body,
        grid=(n // ROWB,),
        in_specs=[
            pl.BlockSpec((ROWB, 128), lambda i: (i, 0)),
            full((1, 128)), full((1, 128)),
            full((128, 256)), full((1, 256)),
            full((256, 128)), full((1, 128)),
        ],
        out_specs=pl.BlockSpec((ROWB, 128), lambda i: (i, 0)),
        out_shape=jax.ShapeDtypeStruct((n, 128), jnp.float32),
    )(x, r2(p["ln_g"]), r2(p["ln_b"]), p["w1"].T, r2(p["b1"]),
      p["w2"].T, r2(p["b2"]))


def _sage_body(agg_ref, cnt_ref, h_ref, wl_ref, bl_ref, wr_ref, o_ref):
    cc = cnt_ref[...]
    cnt = cc[0, :, :1] + cc[1, :, :1]
    mean = agg_ref[...] * (1.0 / jnp.maximum(cnt, 1.0))
    o = (
        jnp.dot(mean, wl_ref[...], preferred_element_type=jnp.float32)
        + bl_ref[...]
        + jnp.dot(h_ref[...], wr_ref[...], preferred_element_type=jnp.float32)
    )
    o_ref[...] = _gelu(o)


def _sage(agg, cnt2, h, p, n):
    full = lambda shape: pl.BlockSpec(shape, lambda i: (0, 0))
    return pl.pallas_call(
        _sage_body,
        grid=(n // ROWB,),
        in_specs=[
            pl.BlockSpec((ROWB, 128), lambda i: (i, 0)),
            pl.BlockSpec((2, ROWB, 16), lambda i: (0, i, 0)),
            pl.BlockSpec((ROWB, 128), lambda i: (i, 0)),
            full((128, 128)), full((1, 128)), full((128, 128)),
        ],
        out_specs=pl.BlockSpec((ROWB, 128), lambda i: (i, 0)),
        out_shape=jax.ShapeDtypeStruct((n, 128), jnp.float32),
    )(agg, cnt2, h, p["wl"].T, p["bl"].reshape(1, -1), p["wr"].T)


_DDN = (((0,), (0,)), ((), ()))


def _pool_ego_body(h_ref, b_ref, f_ref, s_ref, c_ref, ps_ref, cn_ref, mx_ref):
    ph = pl.program_id(0)
    i = pl.program_id(1)
    bat = b_ref[...]
    frm = f_ref[...]
    bcol = bat[:, :64]
    iot = lax.broadcasted_iota(jnp.int32, (ROWB, 64), 1)
    mask = bcol == iot

    @pl.when(jnp.logical_and(ph == 0, i == 0))
    def _():
        mx_ref[...] = jnp.full((8, 64), INT_MIN, jnp.int32)

    @pl.when(ph == 0)
    def _():
        vals = jnp.where(mask, frm[:, :64], INT_MIN)
        bm = jnp.max(vals, axis=0, keepdims=True)
        mx_ref[...] = jnp.maximum(mx_ref[...], jnp.broadcast_to(bm, (8, 64)))

    @pl.when(ph == 1)
    def _():
        maskf = mask.astype(jnp.float32)
        h = h_ref[...]
        mxrow = mx_ref[0:1, :]
        nmax = jnp.sum(
            jnp.where(mask, jnp.broadcast_to(mxrow, (ROWB, 64)), 0),
            axis=1, keepdims=True,
        )
        m = (frm[:, :1] == nmax).astype(jnp.float32)
        dot = lambda a, b: lax.dot_general(
            a, b, _DDN, preferred_element_type=jnp.float32
        )
        s_part = dot(maskf, h * m)
        c_part = dot(maskf, jnp.broadcast_to(m, (ROWB, 128)))
        ps_part = dot(maskf, h)
        cn_part = dot(maskf, jnp.ones((ROWB, 128), jnp.float32))

        @pl.when(i == 0)
        def _():
            s_ref[...] = s_part
            c_ref[...] = c_part
            ps_ref[...] = ps_part
            cn_ref[...] = cn_part

        @pl.when(i > 0)
        def _():
            s_ref[...] += s_part
            c_ref[...] += c_part
            ps_ref[...] += ps_part
            cn_ref[...] += cn_part


def _pool_ego(h, bat, frm):
    o64 = pl.BlockSpec((64, 128), lambda p, i: (0, 0))
    shp = jax.ShapeDtypeStruct((64, 128), jnp.float32)
    return pl.pallas_call(
        _pool_ego_body,
        grid=(2, N_EGO // ROWB),
        in_specs=[
            pl.BlockSpec((ROWB, 128), lambda p, i: (i, 0)),
            pl.BlockSpec((ROWB, 128), lambda p, i: (i, 0)),
            pl.BlockSpec((ROWB, 128), lambda p, i: (i, 0)),
        ],
        out_specs=[o64, o64, o64, o64],
        out_shape=[shp, shp, shp, shp],
        scratch_shapes=[pltpu.VMEM((8, 64), jnp.int32)],
    )(h, bat, frm)


def _pool_veh_body(h_ref, b_ref, ps_ref, cn_ref):
    i = pl.program_id(0)
    bcol = b_ref[...][:, :64]
    iot = lax.broadcasted_iota(jnp.int32, (ROWB, 64), 1)
    maskf = (bcol == iot).astype(jnp.float32)
    dot = lambda a, b: lax.dot_general(
        a, b, _DDN, preferred_element_type=jnp.float32
    )
    ps_part = dot(maskf, h_ref[...])
    cn_part = dot(maskf, jnp.ones((ROWB, 128), jnp.float32))

    @pl.when(i == 0)
    def _():
        ps_ref[...] = ps_part
        cn_ref[...] = cn_part

    @pl.when(i > 0)
    def _():
        ps_ref[...] += ps_part
        cn_ref[...] += cn_part


def _pool_veh(h, bat):
    o64 = pl.BlockSpec((64, 128), lambda i: (0, 0))
    shp = jax.ShapeDtypeStruct((64, 128), jnp.float32)
    return pl.pallas_call(
        _pool_veh_body,
        grid=(N_VEH // ROWB,),
        in_specs=[
            pl.BlockSpec((ROWB, 128), lambda i: (i, 0)),
            pl.BlockSpec((ROWB, 128), lambda i: (i, 0)),
        ],
        out_specs=[o64, o64],
        out_shape=[shp, shp],
    )(h, bat)


def _final_body(s_ref, c_ref, pse_ref, ce_ref, psv_ref, cv_ref,
                rg, rb, rw1, rb1, rw2, rb2,
                hg, hb, hw1, hb1, hw2, hb2,
                y_ref, agg_ref, z_ref):
    s = s_ref[...]
    c = c_ref[...]
    hagg = s / jnp.maximum(c, 1.0)
    agg_ref[...] = hagg

    ce = ce_ref[...]
    cv = cv_ref[...]
    pm_e = pse_ref[...] / jnp.maximum(ce, 1.0)
    pm_v = psv_ref[...] / jnp.maximum(cv, 1.0)
    w_e = (ce > 0).astype(jnp.float32)
    w_v = (cv > 0).astype(jnp.float32)
    z0 = (pm_e * w_e + pm_v * w_v) / jnp.maximum(w_e + w_v, 1.0)

    zn = _ln(z0, rg[...], rb[...])
    zh = _gelu(jnp.dot(zn, rw1[...], preferred_element_type=jnp.float32) + rb1[...])
    z1 = jnp.dot(zh, rw2[...], preferred_element_type=jnp.float32) + rb2[...]
    nrm = jnp.sqrt(jnp.sum(z1 * z1, axis=1, keepdims=True))
    z_ref[...] = z1 / jnp.maximum(nrm, 1e-12)

    yn = _ln(hagg, hg[...], hb[...])
    yh = _gelu(jnp.dot(yn, hw1[...], preferred_element_type=jnp.float32) + hb1[...])
    y_ref[...] = jnp.dot(yh, hw2[...], preferred_element_type=jnp.float32) + hb2[...]


def _final(s, c, ps_e, ce, ps_v, cv, pr, ph):
    r2 = lambda a: a.reshape(1, -1)
    hw1 = jnp.pad(ph["w1"].T, ((0, 0), (0, 128 - ph["w1"].shape[0])))
    hb1 = jnp.pad(r2(ph["b1"]), ((0, 0), (0, 128 - ph["b1"].shape[0])))
    hw2 = jnp.pad(
        ph["w2"].T,
        ((0, 128 - ph["w2"].shape[1]), (0, 128 - ph["w2"].shape[0])),
    )
    hb2 = jnp.pad(r2(ph["b2"]), ((0, 0), (0, 128 - ph["b2"].shape[0])))
    ins = [s, c, ps_e, ce, ps_v, cv,
           r2(pr["ln_g"]), r2(pr["ln_b"]), pr["w1"].T, r2(pr["b1"]),
           pr["w2"].T, r2(pr["b2"]),
           r2(ph["ln_g"]), r2(ph["ln_b"]), hw1, hb1, hw2, hb2]
    specs = [pl.BlockSpec(a.shape, lambda: (0,) * a.ndim) for a in ins]
    o = lambda shape: pl.BlockSpec(shape, lambda: (0, 0))
    y128, hagg, z = pl.pallas_call(
        _final_body,
        in_specs=specs,
        out_specs=[o((64, 128)), o((64, 128)), o((64, 128))],
        out_shape=[jax.ShapeDtypeStruct((64, 128), jnp.float32)] * 3,
    )(*ins)
    return y128, hagg, z


def kernel(x_ego, x_vehicle, params, src_ve, dst_ve, src_ev, dst_ev,
           batch_ego, batch_vehicle, frame_ego):
    P = params
    h_ego = _enc(x_ego, P["enc_ego"], N_EGO)
    h_veh = _enc(x_vehicle, P["enc_vehicle"], N_VEH)

    srcI_ve, dstI_ve = _pad_edges(src_ve, dst_ve, N_VEH, N_EGO)
    srcI_ev, dstI_ev = _pad_edges(src_ev, dst_ev, N_EGO, N_VEH)

    rpt_e = PADDED[N_EGO] // 16
    rpt_v = PADDED[N_VEH] // 16

    ones16 = jnp.ones((128, 16), jnp.float32)
    ce0, ce1 = _make_sc_count(N_EGO)(
        ones16, dstI_ve, jnp.zeros((rpt_e, 16), jnp.float32))
    cv0, cv1 = _make_sc_count(N_VEH)(
        ones16, dstI_ev, jnp.zeros((rpt_v, 16), jnp.float32))
    cnt2_ego = jnp.stack([ce0[:N_EGO], ce1[:N_EGO]])
    cnt2_veh = jnp.stack([cv0[:N_VEH], cv1[:N_VEH]])

    agg_ve_k = _make_sc_agg(N_VEH, N_EGO, 16, 4)
    agg_ev_k = _make_sc_agg(N_EGO, N_VEH, 16, 4)
    zer_e = jnp.zeros((rpt_e, 16), jnp.float32)
    zer_v = jnp.zeros((rpt_v, 16), jnp.float32)

    def _tab(h, n):
        hp = jnp.pad(h, ((0, PADDED[n] - n), (0, 0)))
        return hp.reshape(PADDED[n], 8, 16).transpose(1, 0, 2).reshape(-1, 16)

    for p in P["convs"]:
        tab_veh = _tab(h_veh, N_VEH)
        tab_ego = _tab(h_ego, N_EGO)
        outs_e = agg_ve_k(tab_veh, srcI_ve, dstI_ve, zer_e)
        outs_v = agg_ev_k(tab_ego, srcI_ev, dstI_ev, zer_v)
        agg_ego = jnp.concatenate([o[:N_EGO] for o in outs_e], axis=1)
        agg_veh = jnp.concatenate([o[:N_VEH] for o in outs_v], axis=1)
        h_ego_n = _sage(agg_ego, cnt2_ego, h_ego, p["ve"], N_EGO)
        h_veh_n = _sage(agg_veh, cnt2_veh, h_veh, p["ev"], N_VEH)
        h_ego, h_veh = h_ego_n, h_veh_n

    be = jnp.broadcast_to(batch_ego[:, None], (N_EGO, 128))
    fe = jnp.broadcast_to(frame_ego[:, None], (N_EGO, 128))
    bv = jnp.broadcast_to(batch_vehicle[:, None], (N_VEH, 128))
    s, c, ps_e, ce = _pool_ego(h_ego, be, fe)
    ps_v, cv = _pool_veh(h_veh, bv)
    y128, hagg, z = _final(s, c, ps_e, ce, ps_v, cv, P["readout"], P["head"])
    return (y128[:, :10], hagg, z)


# speedup vs baseline: 1.4948x; 1.4948x over previous
import functools

import jax
import jax.numpy as jnp
from jax import lax
from jax.experimental import pallas as pl
from jax.experimental.pallas import tpu as pltpu
from jax.experimental.pallas import tpu_sc as plsc

N_EGO = 10000
N_VEH = 40000
E = 250000
HID = 128
B = 64

PAD_ROWS = 64
EPAD = 262144
PADDED = {N_EGO: 10112, N_VEH: 40064}
NB = 4
ROWB = 1000
INT_MIN = -2147483648
_INV_SQRT2 = 0.7071067811865476


def _gelu(x):
    return 0.5 * x * (1.0 + lax.erf(x * _INV_SQRT2))


def _ln(x, g, b):
    mu = jnp.mean(x, axis=1, keepdims=True)
    xc = x - mu
    var = jnp.mean(xc * xc, axis=1, keepdims=True)
    return xc * lax.rsqrt(var + 1e-5) * g + b


@functools.lru_cache(maxsize=None)
def _make_sc_agg(n_src, n_dst, dc, ppc, CH=128, NBA=NB):
    n_dst_pad = PADDED[n_dst]
    rpt = n_dst_pad // 16
    n_q = 2 * ppc
    chunks = EPAD // 16 // CH
    mesh = plsc.VectorSubcoreMesh(
        core_axis_name="c", subcore_axis_name="s", num_cores=2, num_subcores=16
    )
    out_type = tuple(
        jax.ShapeDtypeStruct((n_dst_pad, dc), jnp.float32) for _ in range(n_q)
    )
    scratch = [
        pltpu.VMEM((chunks * CH,), jnp.int32),
        pltpu.VMEM((chunks, CH), jnp.int32),
        pltpu.VMEM((NBA, CH, dc), jnp.float32),
        pltpu.VMEM_SHARED((n_dst_pad, dc), jnp.float32),
        pltpu.SemaphoreType.DMA((NBA,)),
        pltpu.SemaphoreType.DMA((NBA,)),
    ]

    def body(table, srcI, dstI, zeros, *rest):
        outs = rest[:n_q]
        sidx, didx, gbuf, acc, gsem, ssem = rest[n_q:]
        c = lax.axis_index("c")
        s = lax.axis_index("s")
        pltpu.sync_copy(srcI.at[pl.ds(s * (chunks * CH), chunks * CH)], sidx)
        pltpu.sync_copy(dstI.at[pl.ds(s * chunks, chunks)], didx)

        for p in range(ppc):
            pltpu.sync_copy(zeros, acc.at[pl.ds(s * rpt, rpt)])
            delta = c * (ppc * n_src) if p == 0 else n_src

            def addoff(i, _, delta=delta):
                sidx[pl.ds(i * 16, 16)] = sidx[pl.ds(i * 16, 16)] + delta
                return 0

            lax.fori_loop(0, chunks * (CH // 16), addoff, 0)
            plsc.subcore_barrier()

            def g_copy(k, b):
                return pltpu.make_async_copy(
                    table.at[sidx.at[pl.ds(k * CH, CH)]], gbuf.at[b],
                    gsem.at[b],
                )

            def s_copy(k, b):
                return pltpu.make_async_copy(
                    gbuf.at[b], acc.at[didx.at[k]], ssem.at[b]
                )

            for b in range(NBA):
                g_copy(b, b).start()

            def outer(j, _):
                for b in range(NBA):
                    k = j * NBA + b
                    g_copy(k, b).wait()
                    s_copy(k, b).start(add=True)
                for b in range(NBA):
                    k = j * NBA + b
                    s_copy(k, b).wait()

                    @pl.when(k + NBA < chunks)
                    def _(k=k, b=b):
                        g_copy(k + NBA, b).start()

                return 0

            lax.fori_loop(0, chunks // NBA, outer, 0)
            tail = chunks % NBA
            for t in range(tail):
                k = chunks - tail + t
                b = k % NBA
                g_copy(k, b).wait()
                s_copy(k, b).start(add=True)
                s_copy(k, b).wait()
            plsc.subcore_barrier()

            for qo in range(n_q):
                if qo % ppc != p:
                    continue

                @pl.when(c == qo // ppc)
                def _(qo=qo):
                    pltpu.sync_copy(
                        acc.at[pl.ds(s * rpt, rpt)],
                        outs[qo].at[pl.ds(s * rpt, rpt)],
                    )

    return pl.kernel(
        body, out_type=out_type, mesh=mesh, scratch_types=scratch,
        compiler_params=pltpu.CompilerParams(use_tc_tiling_on_sc=False),
    )


@functools.lru_cache(maxsize=None)
def _make_sc_count(n_dst):
    n_dst_pad = PADDED[n_dst]
    rpt = n_dst_pad // 16
    chunks = 64
    mesh = plsc.VectorSubcoreMesh(
        core_axis_name="c", subcore_axis_name="s", num_cores=2, num_subcores=16
    )
    out_type = (
        jax.ShapeDtypeStruct((n_dst_pad, 16), jnp.float32),
        jax.ShapeDtypeStruct((n_dst_pad, 16), jnp.float32),
    )
    scratch = [
        pltpu.VMEM((chunks, 128), jnp.int32),
        pltpu.VMEM((128, 16), jnp.float32),
        pltpu.VMEM_SHARED((n_dst_pad, 16), jnp.float32),
        pltpu.SemaphoreType.DMA((NB,)),
    ]

    def body(ones, dstI, zeros, out0, out1, didx, gbuf, acc, ssem):
        c = lax.axis_index("c")
        s = lax.axis_index("s")
        w = c * 16 + s
        pltpu.sync_copy(dstI.at[pl.ds(w * chunks, chunks)], didx)
        pltpu.sync_copy(ones, gbuf)
        pltpu.sync_copy(zeros, acc.at[pl.ds(s * rpt, rpt)])
        plsc.subcore_barrier()

        def s_copy(k, b):
            return pltpu.make_async_copy(gbuf, acc.at[didx.at[k]], ssem.at[b])

        def outer(j, _):
            for b in range(NB):
                s_copy(j * NB + b, b).start(add=True)
            for b in range(NB):
                s_copy(j * NB + b, b).wait()
            return 0

        lax.fori_loop(0, chunks // NB, outer, 0)
        plsc.subcore_barrier()

        outs = (out0, out1)
        for oc in range(2):
            @pl.when(c == oc)
            def _(oc=oc):
                pltpu.sync_copy(
                    acc.at[pl.ds(s * rpt, rpt)],
                    outs[oc].at[pl.ds(s * rpt, rpt)],
                )

    return pl.kernel(
        body, out_type=out_type, mesh=mesh, scratch_types=scratch,
        compiler_params=pltpu.CompilerParams(use_tc_tiling_on_sc=False),
    )


def _pad_edges(src, dst, n_src, n_dst):
    pad = EPAD - E
    ar = jnp.arange(pad, dtype=jnp.int32)
    src_p = jnp.concatenate([src.astype(jnp.int32), ar % n_src])
    dst_p = jnp.concatenate(
        [dst.astype(jnp.int32), n_dst + (ar % PAD_ROWS)]
    )
    return src_p, dst_p.reshape(EPAD // 128, 128)


def _enc_body(x_ref, g_ref, b_ref, w1_ref, b1_ref, w2_ref, b2_ref, o_ref):
    x = x_ref[...]
    xn = _ln(x, g_ref[...], b_ref[...])
    h = jnp.dot(xn, w1_ref[...], preferred_element_type=jnp.float32) + b1_ref[...]
    h = _gelu(h)
    o_ref[...] = (
        jnp.dot(h, w2_ref[...], preferred_element_type=jnp.float32) + b2_ref[...]
    )


def _enc(x, p, n):
    r2 = lambda a: a.reshape(1, -1)
    full = lambda shape: pl.BlockSpec(shape, lambda i: (0, 0))
    return pl.pallas_call(
        _enc_body,
        grid=(n // ROWB,),
        in_specs=[
            pl.BlockSpec((ROWB, 128), lambda i: (i, 0)),
            full((1, 128)), full((1, 128)),
            full((128, 256)), full((1, 256)),
            full((256, 128)), full((1, 128)),
        ],
        out_specs=pl.BlockSpec((ROWB, 128), lambda i: (i, 0)),
        out_shape=jax.ShapeDtypeStruct((n, 128), jnp.float32),
    )(x, r2(p["ln_g"]), r2(p["ln_b"]), p["w1"].T, r2(p["b1"]),
      p["w2"].T, r2(p["b2"]))


def _sage_body(agg_ref, cnt_ref, h_ref, wl_ref, bl_ref, wr_ref, o_ref):
    cc = cnt_ref[...]
    cnt = cc[0, :, :1] + cc[1, :, :1]
    mean = agg_ref[...] * (1.0 / jnp.maximum(cnt, 1.0))
    o = (
        jnp.dot(mean, wl_ref[...], preferred_element_type=jnp.float32)
        + bl_ref[...]
        + jnp.dot(h_ref[...], wr_ref[...], preferred_element_type=jnp.float32)
    )
    o_ref[...] = _gelu(o)


def _sage(agg, cnt2, h, p, n):
    full = lambda shape: pl.BlockSpec(shape, lambda i: (0, 0))
    return pl.pallas_call(
        _sage_body,
        grid=(n // ROWB,),
        in_specs=[
            pl.BlockSpec((ROWB, 128), lambda i: (i, 0)),
            pl.BlockSpec((2, ROWB, 16), lambda i: (0, i, 0)),
            pl.BlockSpec((ROWB, 128), lambda i: (i, 0)),
            full((128, 128)), full((1, 128)), full((128, 128)),
        ],
        out_specs=pl.BlockSpec((ROWB, 128), lambda i: (i, 0)),
        out_shape=jax.ShapeDtypeStruct((n, 128), jnp.float32),
    )(agg, cnt2, h, p["wl"].T, p["bl"].reshape(1, -1), p["wr"].T)


_DDN = (((0,), (0,)), ((), ()))


def _pool_ego_body(h_ref, b_ref, f_ref, s_ref, c_ref, ps_ref, cn_ref, mx_ref):
    ph = pl.program_id(0)
    i = pl.program_id(1)
    bat = b_ref[...]
    frm = f_ref[...]
    bcol = bat[:, :64]
    iot = lax.broadcasted_iota(jnp.int32, (ROWB, 64), 1)
    mask = bcol == iot

    @pl.when(jnp.logical_and(ph == 0, i == 0))
    def _():
        mx_ref[...] = jnp.full((8, 64), INT_MIN, jnp.int32)

    @pl.when(ph == 0)
    def _():
        vals = jnp.where(mask, frm[:, :64], INT_MIN)
        bm = jnp.max(vals, axis=0, keepdims=True)
        mx_ref[...] = jnp.maximum(mx_ref[...], jnp.broadcast_to(bm, (8, 64)))

    @pl.when(ph == 1)
    def _():
        maskf = mask.astype(jnp.float32)
        h = h_ref[...]
        mxrow = mx_ref[0:1, :]
        nmax = jnp.sum(
            jnp.where(mask, jnp.broadcast_to(mxrow, (ROWB, 64)), 0),
            axis=1, keepdims=True,
        )
        m = (frm[:, :1] == nmax).astype(jnp.float32)
        dot = lambda a, b: lax.dot_general(
            a, b, _DDN, preferred_element_type=jnp.float32
        )
        s_part = dot(maskf, h * m)
        c_part = dot(maskf, jnp.broadcast_to(m, (ROWB, 128)))
        ps_part = dot(maskf, h)
        cn_part = dot(maskf, jnp.ones((ROWB, 128), jnp.float32))

        @pl.when(i == 0)
        def _():
            s_ref[...] = s_part
            c_ref[...] = c_part
            ps_ref[...] = ps_part
            cn_ref[...] = cn_part

        @pl.when(i > 0)
        def _():
            s_ref[...] += s_part
            c_ref[...] += c_part
            ps_ref[...] += ps_part
            cn_ref[...] += cn_part


def _pool_ego(h, bat, frm):
    o64 = pl.BlockSpec((64, 128), lambda p, i: (0, 0))
    shp = jax.ShapeDtypeStruct((64, 128), jnp.float32)
    return pl.pallas_call(
        _pool_ego_body,
        grid=(2, N_EGO // ROWB),
        in_specs=[
            pl.BlockSpec((ROWB, 128), lambda p, i: (i, 0)),
            pl.BlockSpec((ROWB, 128), lambda p, i: (i, 0)),
            pl.BlockSpec((ROWB, 128), lambda p, i: (i, 0)),
        ],
        out_specs=[o64, o64, o64, o64],
        out_shape=[shp, shp, shp, shp],
        scratch_shapes=[pltpu.VMEM((8, 64), jnp.int32)],
    )(h, bat, frm)


def _pool_veh_body(h_ref, b_ref, ps_ref, cn_ref):
    i = pl.program_id(0)
    bcol = b_ref[...][:, :64]
    iot = lax.broadcasted_iota(jnp.int32, (ROWB, 64), 1)
    maskf = (bcol == iot).astype(jnp.float32)
    dot = lambda a, b: lax.dot_general(
        a, b, _DDN, preferred_element_type=jnp.float32
    )
    ps_part = dot(maskf, h_ref[...])
    cn_part = dot(maskf, jnp.ones((ROWB, 128), jnp.float32))

    @pl.when(i == 0)
    def _():
        ps_ref[...] = ps_part
        cn_ref[...] = cn_part

    @pl.when(i > 0)
    def _():
        ps_ref[...] += ps_part
        cn_ref[...] += cn_part


def _pool_veh(h, bat):
    o64 = pl.BlockSpec((64, 128), lambda i: (0, 0))
    shp = jax.ShapeDtypeStruct((64, 128), jnp.float32)
    return pl.pallas_call(
        _pool_veh_body,
        grid=(N_VEH // ROWB,),
        in_specs=[
            pl.BlockSpec((ROWB, 128), lambda i: (i, 0)),
            pl.BlockSpec((ROWB, 128), lambda i: (i, 0)),
        ],
        out_specs=[o64, o64],
        out_shape=[shp, shp],
    )(h, bat)


def _final_body(s_ref, c_ref, pse_ref, ce_ref, psv_ref, cv_ref,
                rg, rb, rw1, rb1, rw2, rb2,
                hg, hb, hw1, hb1, hw2, hb2,
                y_ref, agg_ref, z_ref):
    s = s_ref[...]
    c = c_ref[...]
    hagg = s / jnp.maximum(c, 1.0)
    agg_ref[...] = hagg

    ce = ce_ref[...]
    cv = cv_ref[...]
    pm_e = pse_ref[...] / jnp.maximum(ce, 1.0)
    pm_v = psv_ref[...] / jnp.maximum(cv, 1.0)
    w_e = (ce > 0).astype(jnp.float32)
    w_v = (cv > 0).astype(jnp.float32)
    z0 = (pm_e * w_e + pm_v * w_v) / jnp.maximum(w_e + w_v, 1.0)

    zn = _ln(z0, rg[...], rb[...])
    zh = _gelu(jnp.dot(zn, rw1[...], preferred_element_type=jnp.float32) + rb1[...])
    z1 = jnp.dot(zh, rw2[...], preferred_element_type=jnp.float32) + rb2[...]
    nrm = jnp.sqrt(jnp.sum(z1 * z1, axis=1, keepdims=True))
    z_ref[...] = z1 / jnp.maximum(nrm, 1e-12)

    yn = _ln(hagg, hg[...], hb[...])
    yh = _gelu(jnp.dot(yn, hw1[...], preferred_element_type=jnp.float32) + hb1[...])
    y_ref[...] = jnp.dot(yh, hw2[...], preferred_element_type=jnp.float32) + hb2[...]


def _final(s, c, ps_e, ce, ps_v, cv, pr, ph):
    r2 = lambda a: a.reshape(1, -1)
    hw1 = jnp.pad(ph["w1"].T, ((0, 0), (0, 128 - ph["w1"].shape[0])))
    hb1 = jnp.pad(r2(ph["b1"]), ((0, 0), (0, 128 - ph["b1"].shape[0])))
    hw2 = jnp.pad(
        ph["w2"].T,
        ((0, 128 - ph["w2"].shape[1]), (0, 128 - ph["w2"].shape[0])),
    )
    hb2 = jnp.pad(r2(ph["b2"]), ((0, 0), (0, 128 - ph["b2"].shape[0])))
    ins = [s, c, ps_e, ce, ps_v, cv,
           r2(pr["ln_g"]), r2(pr["ln_b"]), pr["w1"].T, r2(pr["b1"]),
           pr["w2"].T, r2(pr["b2"]),
           r2(ph["ln_g"]), r2(ph["ln_b"]), hw1, hb1, hw2, hb2]
    specs = [pl.BlockSpec(a.shape, lambda: (0,) * a.ndim) for a in ins]
    o = lambda shape: pl.BlockSpec(shape, lambda: (0, 0))
    y128, hagg, z = pl.pallas_call(
        _final_body,
        in_specs=specs,
        out_specs=[o((64, 128)), o((64, 128)), o((64, 128))],
        out_shape=[jax.ShapeDtypeStruct((64, 128), jnp.float32)] * 3,
    )(*ins)
    return y128, hagg, z


def kernel(x_ego, x_vehicle, params, src_ve, dst_ve, src_ev, dst_ev,
           batch_ego, batch_vehicle, frame_ego):
    P = params
    h_ego = _enc(x_ego, P["enc_ego"], N_EGO)
    h_veh = _enc(x_vehicle, P["enc_vehicle"], N_VEH)

    srcI_ve, dstI_ve = _pad_edges(src_ve, dst_ve, N_VEH, N_EGO)
    srcI_ev, dstI_ev = _pad_edges(src_ev, dst_ev, N_EGO, N_VEH)

    rpt_e = PADDED[N_EGO] // 16
    rpt_v = PADDED[N_VEH] // 16

    ones16 = jnp.ones((128, 16), jnp.float32)
    ce0, ce1 = _make_sc_count(N_EGO)(
        ones16, dstI_ve, jnp.zeros((rpt_e, 16), jnp.float32))
    cv0, cv1 = _make_sc_count(N_VEH)(
        ones16, dstI_ev, jnp.zeros((rpt_v, 16), jnp.float32))
    cnt2_ego = jnp.stack([ce0[:N_EGO], ce1[:N_EGO]])
    cnt2_veh = jnp.stack([cv0[:N_VEH], cv1[:N_VEH]])

    agg_ve_k = _make_sc_agg(N_VEH, N_EGO, 64, 1)
    agg_ev_k = _make_sc_agg(N_EGO, N_VEH, 32, 2)
    zer_e = jnp.zeros((rpt_e, 64), jnp.float32)
    zer_v = jnp.zeros((rpt_v, 32), jnp.float32)

    for p in P["convs"]:
        tab_veh = h_veh.reshape(N_VEH, 2, 64).transpose(1, 0, 2).reshape(-1, 64)
        tab_ego = h_ego.reshape(N_EGO, 4, 32).transpose(1, 0, 2).reshape(-1, 32)
        outs_e = agg_ve_k(tab_veh, srcI_ve, dstI_ve, zer_e)
        outs_v = agg_ev_k(tab_ego, srcI_ev, dstI_ev, zer_v)
        agg_ego = jnp.concatenate([o[:N_EGO] for o in outs_e], axis=1)
        agg_veh = jnp.concatenate([o[:N_VEH] for o in outs_v], axis=1)
        h_ego_n = _sage(agg_ego, cnt2_ego, h_ego, p["ve"], N_EGO)
        h_veh_n = _sage(agg_veh, cnt2_veh, h_veh, p["ev"], N_VEH)
        h_ego, h_veh = h_ego_n, h_veh_n

    be = jnp.broadcast_to(batch_ego[:, None], (N_EGO, 128))
    fe = jnp.broadcast_to(frame_ego[:, None], (N_EGO, 128))
    bv = jnp.broadcast_to(batch_vehicle[:, None], (N_VEH, 128))
    s, c, ps_e, ce = _pool_ego(h_ego, be, fe)
    ps_v, cv = _pool_veh(h_veh, bv)
    y128, hagg, z = _final(s, c, ps_e, ce, ps_v, cv, P["readout"], P["head"])
    return (y128[:, :10], hagg, z)
